# Initial kernel scaffold; baseline (speedup 1.0000x reference)
import jax
import jax.numpy as jnp
from jax.experimental import pallas as pl


def kernel(inputs, embed_bow, embed_bigram, W1, b1, W2, b2):
    raise NotImplementedError("write your pallas kernel here")


# speedup vs baseline: 1.6601x; 1.6601x over previous
import functools

import jax
import jax.numpy as jnp
from jax import lax
from jax.experimental import pallas as pl
from jax.experimental.pallas import tpu as pltpu
from jax.experimental.pallas import tpu_sc as plsc

VOCAB = 1000000
EMBED = 64
B = 4096
L = 200
NCLS = 5

NUM_CORES = 2
NUM_SUBCORES = 16
NW = NUM_CORES * NUM_SUBCORES
BPW = B // NW
WAVE = 8


def _pool_sums(idx_t, embed_bow, embed_bigram):
    mesh = plsc.VectorSubcoreMesh(
        core_axis_name="c", subcore_axis_name="s",
        num_cores=NUM_CORES, num_subcores=NUM_SUBCORES)

    @functools.partial(
        pl.kernel,
        out_type=jax.ShapeDtypeStruct((2, B, EMBED), jnp.float32),
        mesh=mesh,
        scratch_types=[
            pltpu.VMEM((L, BPW), jnp.int32),
            pltpu.VMEM((BPW, EMBED), jnp.float32),
            pltpu.SemaphoreType.DMA,
        ],
        compiler_params=pltpu.CompilerParams(use_tc_tiling_on_sc=False),
    )
    def pool(idx_hbm, bow_hbm, big_hbm, out_hbm, idx_v, acc_v, sem):
        wid = lax.axis_index("c") * NUM_SUBCORES + lax.axis_index("s")
        base = wid * BPW
        zeros = jnp.zeros((16,), jnp.float32)
        for t, tab in ((0, bow_hbm), (1, big_hbm)):
            pltpu.sync_copy(idx_hbm.at[t, :, pl.ds(base, BPW)], idx_v)

            @pl.loop(0, BPW)
            def _zero(i):
                for j in range(EMBED // 16):
                    acc_v[i, pl.ds(j * 16, 16)] = zeros

            @pl.loop(0, L, step=WAVE)
            def _wave(l0):
                cps = [
                    pltpu.async_copy(tab.at[idx_v.at[l0 + j]], acc_v, sem,
                                     add=True)
                    for j in range(WAVE)
                ]
                for cp in cps:
                    cp.wait()

            pltpu.sync_copy(acc_v, out_hbm.at[t, pl.ds(base, BPW), :])

    return pool(idx_t, embed_bow, embed_bigram)


def _mlp_body(f_ref, w1_ref, b1_ref, w2_ref, b2_ref, o_ref):
    inv_l = 1.0 / L
    x0 = f_ref[0] * inv_l
    x1 = f_ref[1] * inv_l
    w1 = w1_ref[...]
    h = (lax.dot_general(x0, w1[:, :EMBED], (((1,), (1,)), ((), ())),
                         preferred_element_type=jnp.float32)
         + lax.dot_general(x1, w1[:, EMBED:], (((1,), (1,)), ((), ())),
                           preferred_element_type=jnp.float32))
    h = jnp.maximum(h + b1_ref[...], 0.0)
    logits = lax.dot_general(h, w2_ref[...], (((1,), (1,)), ((), ())),
                             preferred_element_type=jnp.float32)
    logits = logits + b2_ref[...]
    m = jnp.max(logits, axis=1, keepdims=True)
    e = jnp.exp(logits - m)
    o_ref[...] = e / jnp.sum(e, axis=1, keepdims=True)


def kernel(inputs, embed_bow, embed_bigram, W1, b1, W2, b2):
    idx_t = jnp.transpose(inputs.astype(jnp.int32), (0, 2, 1))
    feat = _pool_sums(idx_t, embed_bow, embed_bigram)
    out = pl.pallas_call(
        _mlp_body,
        out_shape=jax.ShapeDtypeStruct((B, NCLS), jnp.float32),
    )(feat, W1, b1.reshape(1, EMBED), W2, b2.reshape(1, NCLS))
    return out
